# baseline (speedup 1.0000x reference)
import functools

import jax
import jax.numpy as jnp
from jax import lax
from jax.experimental import pallas as pl
from jax.experimental.pallas import tpu as pltpu
from jax.experimental.pallas import tpu_sc as plsc

N_USERS = 1000000
N_ITEMS = 1000000
EMB = 32
BATCH = 16384

_INFO = plsc.get_sparse_core_info()
_NC = _INFO.num_cores
_NS = _INFO.num_subcores
_L = _INFO.num_lanes
_NW = _NC * _NS
_BPW = BATCH // _NW
_IDX_CHUNK = 128
_NCHUNK = _BPW // _IDX_CHUNK


def _mf_kernel(u_id_hbm, i_id_hbm, uf_hbm, if_hbm, out_hbm,
               uidx_v, iidx_v, urows_v, irows_v, out_v, sem):
    wid = lax.axis_index("s") * _NC + lax.axis_index("c")
    base = wid * _BPW

    for c in range(_NCHUNK):
        pltpu.sync_copy(u_id_hbm.at[pl.ds(base + c * _IDX_CHUNK, _IDX_CHUNK)],
                        uidx_v.at[c])
        pltpu.sync_copy(i_id_hbm.at[pl.ds(base + c * _IDX_CHUNK, _IDX_CHUNK)],
                        iidx_v.at[c])

    copies = []
    for c in range(_NCHUNK):
        copies.append(pltpu.async_copy(
            uf_hbm.at[uidx_v.at[c]],
            urows_v.at[pl.ds(c * _IDX_CHUNK, _IDX_CHUNK)], sem))
        copies.append(pltpu.async_copy(
            if_hbm.at[iidx_v.at[c]],
            irows_v.at[pl.ds(c * _IDX_CHUNK, _IDX_CHUNK)], sem))
    for cp in copies:
        cp.wait()

    def body(chunk, carry):
        rows = lax.iota(jnp.int32, _L) + chunk * _L
        acc = jnp.zeros((_L,), jnp.float32)
        for e in range(EMB):
            col = jnp.full((_L,), e, jnp.int32)
            u = plsc.load_gather(urows_v, [rows, col])
            v = plsc.load_gather(irows_v, [rows, col])
            acc = acc + u * v
        out_v[pl.ds(chunk * _L, _L)] = acc
        return carry

    lax.fori_loop(0, _BPW // _L, body, 0, unroll=False)

    pltpu.sync_copy(out_v, out_hbm.at[pl.ds(base, _BPW)])


@functools.partial(jax.jit)
def kernel(u_id, i_id, user_factors, item_factors):
    u_id = u_id.astype(jnp.int32)
    i_id = i_id.astype(jnp.int32)
    mesh = plsc.VectorSubcoreMesh(core_axis_name="c", subcore_axis_name="s")
    run = pl.kernel(
        _mf_kernel,
        mesh=mesh,
        out_type=jax.ShapeDtypeStruct((BATCH,), jnp.float32),
        scratch_types=[
            pltpu.VMEM((_NCHUNK, _IDX_CHUNK), jnp.int32),
            pltpu.VMEM((_NCHUNK, _IDX_CHUNK), jnp.int32),
            pltpu.VMEM((_BPW, EMB), jnp.float32),
            pltpu.VMEM((_BPW, EMB), jnp.float32),
            pltpu.VMEM((_BPW,), jnp.float32),
            pltpu.SemaphoreType.DMA,
        ],
        compiler_params=pltpu.CompilerParams(
            needs_layout_passes=False, use_tc_tiling_on_sc=False),
    )
    return run(u_id, i_id, user_factors, item_factors)
